# baseline (speedup 1.0000x reference)
import functools

import jax
import jax.numpy as jnp
from jax import lax
from jax.experimental import pallas as pl
from jax.experimental.pallas import tpu as pltpu
from jax.experimental.pallas import tpu_sc as plsc

N_USERS = 10000
N_ENTITIES = 10000
CHANNEL = 128
CHH = CHANNEL // 2
N_FACTORS = 4
N_RELATIONS = 8
E_TOTAL = 320000
TEMP = 0.2

NC = 2
NS = 16
NW = NC * NS
CH = 128
RPW = 80
E_PAD = NW * RPW * CH
CPB = 8
NBLK_TOT = E_PAD // CH // CPB
Q0 = 5
Q1 = (NBLK_TOT - NS * Q0) // NS
NBUF = 8
TRASH_N = 512
ACC_N = N_ENTITIES + TRASH_N
ROWS_PER_TILE = N_ENTITIES // NS
ZROWS = 125


def _sc_hop_body(do_cnt, tlo_hbm, thi_hbm, elo_hbm, ehi_hbm, kgi_hbm,
                 head_hbm, col_hbm, row_hbm, val_hbm,
                 kglo_out, kghi_out, cnt_out, uslo_out, ushi_out,
                 rows_v, idx_v, dst_v, val_v, zbuf, cnt_v, acc, sem, sem2):
    cid = lax.axis_index("c")
    sid = lax.axis_index("s")
    wid = cid * NS + sid
    zeros16 = jnp.zeros((16,), jnp.float32)
    ones16 = jnp.ones((16,), jnp.float32)

    def _zz(i, _):
        for q in range(CHH // 16):
            zbuf[i, pl.ds(q * 16, 16)] = zeros16
        return 0
    lax.fori_loop(0, ZROWS, _zz, 0)

    if do_cnt:
        def _zc(i, _):
            cnt_v[pl.ds(i * 16, 16)] = zeros16
            return 0
        lax.fori_loop(0, ACC_N // 16, _zc, 0)

    def _zero_acc():
        for z in range(ROWS_PER_TILE // ZROWS):
            pltpu.sync_copy(
                zbuf, acc.at[pl.ds(sid * ROWS_PER_TILE + z * ZROWS, ZROWS)])

    def _scatter_pass(table, gidx_hbm, dest_hbm, scale, count):
        def _blk_at(blk0, b):
            r0 = (blk0 + b) * CPB
            i_d = pltpu.async_copy(gidx_hbm.at[pl.ds(r0, CPB)], idx_v, sem)
            d_d = pltpu.async_copy(dest_hbm.at[pl.ds(r0, CPB)], dst_v, sem)
            if scale:
                pltpu.async_copy(val_hbm.at[pl.ds(r0, CPB)],
                                 val_v, sem).wait()
            d_d.wait()
            i_d.wait()
            gd = [pltpu.async_copy(table.at[idx_v.at[k]], rows_v.at[k], sem)
                  for k in range(CPB)]
            sd = [None] * CPB
            for k in range(CPB):
                gd[k].wait()
                if scale:
                    def _scale16(j, _, k=k):
                        v16 = val_v[k, pl.ds(j * 16, 16)]
                        for ee in range(16):
                            e = j * 16 + ee
                            vb = jnp.full((16,), v16[ee], jnp.float32)
                            for q in range(CHH // 16):
                                rows_v[k, e, pl.ds(q * 16, 16)] = (
                                    rows_v[k, e, pl.ds(q * 16, 16)] * vb)
                        return 0
                    lax.fori_loop(0, CH // 16, _scale16, 0)
                sd[k] = pltpu.async_copy(rows_v.at[k], acc.at[dst_v.at[k]],
                                         sem2, add=True)
                if count:
                    def _cnt(j, _, k=k):
                        h16 = dst_v[k, pl.ds(j * 16, 16)]
                        plsc.addupdate_scatter(cnt_v, [h16], ones16)
                        return 0
                    lax.fori_loop(0, CH // 16, _cnt, 0)
            for k in range(CPB):
                sd[k].wait()

        @pl.when(cid == 0)
        def _():
            def _b0(b, _):
                _blk_at(sid * Q0, b)
                return 0
            lax.fori_loop(0, Q0, _b0, 0)

        @pl.when(cid != 0)
        def _():
            def _b1(b, _):
                _blk_at(NS * Q0 + sid * Q1, b)
                return 0
            lax.fori_loop(0, Q1, _b1, 0)

    passes = [
        (tlo_hbm, kgi_hbm, head_hbm, False, do_cnt, kglo_out),
        (thi_hbm, kgi_hbm, head_hbm, False, False, kghi_out),
        (elo_hbm, col_hbm, row_hbm, True, False, uslo_out),
        (ehi_hbm, col_hbm, row_hbm, True, False, ushi_out),
    ]
    for table, gidx, dest, scale, count, out in passes:
        _zero_acc()
        plsc.subcore_barrier()
        _scatter_pass(table, gidx, dest, scale, count)
        plsc.subcore_barrier()
        pltpu.sync_copy(acc.at[pl.ds(sid * ROWS_PER_TILE, ROWS_PER_TILE)],
                        out.at[cid, sid])
        if count:
            pltpu.sync_copy(cnt_v.at[pl.ds(0, N_ENTITIES)],
                            cnt_out.at[wid, 0])
        plsc.subcore_barrier()


@functools.cache
def _sc_hop_build(do_cnt):
    half = jax.ShapeDtypeStruct((NC, NS, ROWS_PER_TILE, CHH), jnp.float32)
    return pl.kernel(
        functools.partial(_sc_hop_body, do_cnt),
        out_type=(
            half,
            half,
            jax.ShapeDtypeStruct((NW, 1, N_ENTITIES if do_cnt else 16),
                                 jnp.float32),
            half,
            half,
        ),
        mesh=plsc.VectorSubcoreMesh(core_axis_name="c", subcore_axis_name="s",
                                    num_cores=NC, num_subcores=NS),
        compiler_params=pltpu.CompilerParams(use_tc_tiling_on_sc=False,
                                             needs_layout_passes=False),
        scratch_types=[
            pltpu.VMEM((NBUF, CH, CHH), jnp.float32),
            pltpu.VMEM((CPB, CH), jnp.int32),
            pltpu.VMEM((CPB, CH), jnp.int32),
            pltpu.VMEM((CPB, CH), jnp.float32),
            pltpu.VMEM((ZROWS, CHH), jnp.float32),
            pltpu.VMEM((ACC_N,), jnp.float32),
            pltpu.VMEM_SHARED((ACC_N, CHH), jnp.float32),
            pltpu.SemaphoreType.DMA,
            pltpu.SemaphoreType.DMA,
        ],
    )


def _prep_body(ent_ref, w_ref, tlo_ref, thi_ref, elo_ref, ehi_ref):
    r = pl.program_id(0)
    e = ent_ref[...]
    w_row = w_ref[pl.ds(r, 1), :]
    tlo_ref[...] = e[:, :CHH] * w_row[:, :CHH]
    thi_ref[...] = e[:, CHH:] * w_row[:, CHH:]

    @pl.when(r == 0)
    def _():
        elo_ref[...] = e[:, :CHH]
        ehi_ref[...] = e[:, CHH:]


_prep = pl.pallas_call(
    _prep_body,
    grid=(N_RELATIONS - 1,),
    in_specs=[
        pl.BlockSpec((N_ENTITIES, CHANNEL), lambda r: (0, 0)),
        pl.BlockSpec((N_RELATIONS - 1, CHANNEL), lambda r: (0, 0)),
    ],
    out_specs=[
        pl.BlockSpec((N_ENTITIES, CHH), lambda r: (r, 0)),
        pl.BlockSpec((N_ENTITIES, CHH), lambda r: (r, 0)),
        pl.BlockSpec((N_ENTITIES, CHH), lambda r: (0, 0)),
        pl.BlockSpec((N_ENTITIES, CHH), lambda r: (0, 0)),
    ],
    out_shape=[
        jax.ShapeDtypeStruct(((N_RELATIONS - 1) * N_ENTITIES, CHH),
                             jnp.float32),
        jax.ShapeDtypeStruct(((N_RELATIONS - 1) * N_ENTITIES, CHH),
                             jnp.float32),
        jax.ShapeDtypeStruct((N_ENTITIES, CHH), jnp.float32),
        jax.ShapeDtypeStruct((N_ENTITIES, CHH), jnp.float32),
    ],
)


_RB = 1000


def _combine_body(kglo_ref, kghi_ref, cntt_ref, uslo_ref, ushi_ref, usr_ref,
                  lat_ref, att_ref, w_ref, ent_o, usr_o, cor_o):
    sums = jnp.concatenate([kglo_ref[0] + kglo_ref[1],
                            kghi_ref[0] + kghi_ref[1]], axis=1)
    cnt = jnp.sum(cntt_ref[...], axis=1)
    ent_agg = sums / jnp.maximum(cnt, 1.0)[:, None]
    en = jnp.sqrt(jnp.sum(ent_agg * ent_agg, axis=1, keepdims=True))
    ent_o[...] = ent_agg / jnp.maximum(en, 1e-12)

    usr = usr_ref[...]
    logits = lax.dot_general(usr, lat_ref[...], (((1,), (1,)), ((), ())),
                             preferred_element_type=jnp.float32)
    score = jax.nn.softmax(logits, axis=1)
    att = att_ref[...]
    dw = lax.dot_general(jax.nn.softmax(att, axis=-1), w_ref[...],
                         (((1,), (0,)), ((), ())),
                         preferred_element_type=jnp.float32)
    factor = 1.0 + lax.dot_general(score, dw, (((1,), (0,)), ((), ())),
                                   preferred_element_type=jnp.float32)
    us = jnp.concatenate([uslo_ref[0] + uslo_ref[1],
                          ushi_ref[0] + ushi_ref[1]], axis=1)
    usr_agg = us * factor
    un = jnp.sqrt(jnp.sum(usr_agg * usr_agg, axis=1, keepdims=True))
    usr_o[...] = usr_agg / jnp.maximum(un, 1e-12)

    @pl.when(pl.program_id(0) == 0)
    def _():
        gram = lax.dot_general(att, att, (((0,), (0,)), ((), ())),
                               preferred_element_type=jnp.float32)
        ttl = jnp.exp(jnp.sum(gram, axis=1) / TEMP)
        colsq = jnp.sum(att * att, axis=0)
        pos = jnp.exp((colsq / colsq) / TEMP)
        cor_o[...] = (-jnp.sum(jnp.log(pos / ttl))).reshape(1, 1)


_combine = pl.pallas_call(
    _combine_body,
    grid=(N_ENTITIES // _RB,),
    in_specs=[
        pl.BlockSpec((NC, _RB, CHH), lambda i: (0, i, 0)),
        pl.BlockSpec((NC, _RB, CHH), lambda i: (0, i, 0)),
        pl.BlockSpec((_RB, NW), lambda i: (i, 0)),
        pl.BlockSpec((NC, _RB, CHH), lambda i: (0, i, 0)),
        pl.BlockSpec((NC, _RB, CHH), lambda i: (0, i, 0)),
        pl.BlockSpec((_RB, CHANNEL), lambda i: (i, 0)),
        pl.BlockSpec((N_FACTORS, CHANNEL), lambda i: (0, 0)),
        pl.BlockSpec((N_FACTORS, N_RELATIONS - 1), lambda i: (0, 0)),
        pl.BlockSpec((N_RELATIONS - 1, CHANNEL), lambda i: (0, 0)),
    ],
    out_specs=[
        pl.BlockSpec((_RB, CHANNEL), lambda i: (i, 0)),
        pl.BlockSpec((_RB, CHANNEL), lambda i: (i, 0)),
        pl.BlockSpec((1, 1), lambda i: (0, 0)),
    ],
    out_shape=[
        jax.ShapeDtypeStruct((N_ENTITIES, CHANNEL), jnp.float32),
        jax.ShapeDtypeStruct((N_USERS, CHANNEL), jnp.float32),
        jax.ShapeDtypeStruct((1, 1), jnp.float32),
    ],
)


def _pad_idx(x, fill=None):
    if fill is None:
        pad = N_ENTITIES + (jnp.arange(E_PAD - E_TOTAL, dtype=x.dtype)
                            % TRASH_N)
    else:
        pad = jnp.full((E_PAD - E_TOTAL,), fill, x.dtype)
    return jnp.concatenate([x, pad]).reshape(E_PAD // CH, CH)


def kernel(user_emb, entity_emb, latent_emb, edge_index, edge_type,
           interact_indices, interact_values, weight, disen_weight_att):
    head = _pad_idx(edge_index[0])
    kg_idx = _pad_idx((edge_type - 1) * N_ENTITIES + edge_index[1], 0)
    row = _pad_idx(interact_indices[0])
    col = _pad_idx(interact_indices[1], 0)
    val = _pad_idx(interact_values, 0.0)

    ent_cur, usr_cur = entity_emb, user_emb
    ent_res, usr_res = entity_emb, user_emb
    cor = None
    cnt_t = None
    for hop in range(2):
        t_lo, t_hi, e_lo, e_hi = _prep(ent_cur, weight)
        kg_lo, kg_hi, cnt_p, us_lo, us_hi = _sc_hop_build(hop == 0)(
            t_lo, t_hi, e_lo, e_hi, kg_idx, head, col, row, val)
        kg_lo = kg_lo.reshape(NC, N_ENTITIES, CHH)
        kg_hi = kg_hi.reshape(NC, N_ENTITIES, CHH)
        us_lo = us_lo.reshape(NC, N_USERS, CHH)
        us_hi = us_hi.reshape(NC, N_USERS, CHH)
        if hop == 0:
            cnt_t = cnt_p.reshape(NW, N_ENTITIES).T
        ent_cur, usr_cur, cor11 = _combine(kg_lo, kg_hi, cnt_t, us_lo, us_hi,
                                           usr_cur, latent_emb,
                                           disen_weight_att, weight)
        if cor is None:
            cor = cor11[0, 0]
        ent_res = ent_res + ent_cur
        usr_res = usr_res + usr_cur
    return (ent_res, usr_res, cor)


# speedup vs baseline: 1.1988x; 1.1988x over previous
import functools

import jax
import jax.numpy as jnp
from jax import lax
from jax.experimental import pallas as pl
from jax.experimental.pallas import tpu as pltpu
from jax.experimental.pallas import tpu_sc as plsc

N_USERS = 10000
N_ENTITIES = 10000
CHANNEL = 128
CHH = CHANNEL // 2
N_FACTORS = 4
N_RELATIONS = 8
E_TOTAL = 320000
TEMP = 0.2

NC = 2
NS = 16
NW = NC * NS
CH = 128
RPW = 80
E_PAD = NW * RPW * CH
CPB = 8
NBLK_TOT = E_PAD // CH // CPB
Q0 = 15
Q1 = (NBLK_TOT - NS * Q0) // NS
NBUF = 8
TRASH_N = 512
ACC_N = N_ENTITIES + TRASH_N
ROWS_PER_TILE = N_ENTITIES // NS
ZROWS = 125


def _sc_hop_body(do_cnt, tlo_hbm, thi_hbm, elo_hbm, ehi_hbm, kgi_hbm,
                 head_hbm, col_hbm, row_hbm, val_hbm,
                 kglo_out, kghi_out, cnt_out, uslo_out, ushi_out,
                 rows_v, idx_v, dst_v, val_v, zbuf, cnt_v, acc, sem, sem2):
    cid = lax.axis_index("c")
    sid = lax.axis_index("s")
    wid = cid * NS + sid
    zeros16 = jnp.zeros((16,), jnp.float32)
    ones16 = jnp.ones((16,), jnp.float32)

    def _zz(i, _):
        for q in range(CHH // 16):
            zbuf[i, pl.ds(q * 16, 16)] = zeros16
        return 0
    lax.fori_loop(0, ZROWS, _zz, 0)

    if do_cnt:
        def _zc(i, _):
            cnt_v[pl.ds(i * 16, 16)] = zeros16
            return 0
        lax.fori_loop(0, ACC_N // 16, _zc, 0)

    def _zero_acc():
        for z in range(ROWS_PER_TILE // ZROWS):
            pltpu.sync_copy(
                zbuf, acc.at[pl.ds(sid * ROWS_PER_TILE + z * ZROWS, ZROWS)])

    def _scatter_pass(table, gidx_hbm, dest_hbm, scale, count):
        def _blk_at(blk0, b):
            r0 = (blk0 + b) * CPB
            i_d = pltpu.async_copy(gidx_hbm.at[pl.ds(r0, CPB)], idx_v, sem)
            d_d = pltpu.async_copy(dest_hbm.at[pl.ds(r0, CPB)], dst_v, sem)
            if scale:
                pltpu.async_copy(val_hbm.at[pl.ds(r0, CPB)],
                                 val_v, sem).wait()
            d_d.wait()
            i_d.wait()
            gd = [pltpu.async_copy(table.at[idx_v.at[k]], rows_v.at[k], sem)
                  for k in range(CPB)]
            sd = [None] * CPB
            for k in range(CPB):
                gd[k].wait()
                if scale:
                    def _scale16(j, _, k=k):
                        v16 = val_v[k, pl.ds(j * 16, 16)]
                        for ee in range(16):
                            e = j * 16 + ee
                            vb = jnp.full((16,), v16[ee], jnp.float32)
                            for q in range(CHH // 16):
                                rows_v[k, e, pl.ds(q * 16, 16)] = (
                                    rows_v[k, e, pl.ds(q * 16, 16)] * vb)
                        return 0
                    lax.fori_loop(0, CH // 16, _scale16, 0)
                sd[k] = pltpu.async_copy(rows_v.at[k], acc.at[dst_v.at[k]],
                                         sem2, add=True)
                if count:
                    def _cnt(j, _, k=k):
                        h16 = dst_v[k, pl.ds(j * 16, 16)]
                        plsc.addupdate_scatter(cnt_v, [h16], ones16)
                        return 0
                    lax.fori_loop(0, CH // 16, _cnt, 0)
            for k in range(CPB):
                sd[k].wait()

        @pl.when(cid == 0)
        def _():
            def _b0(b, _):
                _blk_at(sid * Q0, b)
                return 0
            lax.fori_loop(0, Q0, _b0, 0)

        @pl.when(cid != 0)
        def _():
            def _b1(b, _):
                _blk_at(NS * Q0 + sid * Q1, b)
                return 0
            lax.fori_loop(0, Q1, _b1, 0)

    passes = [
        (tlo_hbm, kgi_hbm, head_hbm, False, do_cnt, kglo_out),
        (thi_hbm, kgi_hbm, head_hbm, False, False, kghi_out),
        (elo_hbm, col_hbm, row_hbm, True, False, uslo_out),
        (ehi_hbm, col_hbm, row_hbm, True, False, ushi_out),
    ]
    for table, gidx, dest, scale, count, out in passes:
        _zero_acc()
        plsc.subcore_barrier()
        _scatter_pass(table, gidx, dest, scale, count)
        plsc.subcore_barrier()
        pltpu.sync_copy(acc.at[pl.ds(sid * ROWS_PER_TILE, ROWS_PER_TILE)],
                        out.at[cid, sid])
        if count:
            pltpu.sync_copy(cnt_v.at[pl.ds(0, N_ENTITIES)],
                            cnt_out.at[wid, 0])
        plsc.subcore_barrier()


@functools.cache
def _sc_hop_build(do_cnt):
    half = jax.ShapeDtypeStruct((NC, NS, ROWS_PER_TILE, CHH), jnp.float32)
    return pl.kernel(
        functools.partial(_sc_hop_body, do_cnt),
        out_type=(
            half,
            half,
            jax.ShapeDtypeStruct((NW, 1, N_ENTITIES if do_cnt else 16),
                                 jnp.float32),
            half,
            half,
        ),
        mesh=plsc.VectorSubcoreMesh(core_axis_name="c", subcore_axis_name="s",
                                    num_cores=NC, num_subcores=NS),
        compiler_params=pltpu.CompilerParams(use_tc_tiling_on_sc=False,
                                             needs_layout_passes=False),
        scratch_types=[
            pltpu.VMEM((NBUF, CH, CHH), jnp.float32),
            pltpu.VMEM((CPB, CH), jnp.int32),
            pltpu.VMEM((CPB, CH), jnp.int32),
            pltpu.VMEM((CPB, CH), jnp.float32),
            pltpu.VMEM((ZROWS, CHH), jnp.float32),
            pltpu.VMEM((ACC_N,), jnp.float32),
            pltpu.VMEM_SHARED((ACC_N, CHH), jnp.float32),
            pltpu.SemaphoreType.DMA,
            pltpu.SemaphoreType.DMA,
        ],
    )


def _prep_body(ent_ref, w_ref, tlo_ref, thi_ref, elo_ref, ehi_ref):
    r = pl.program_id(0)
    e = ent_ref[...]
    w_row = w_ref[pl.ds(r, 1), :]
    tlo_ref[...] = e[:, :CHH] * w_row[:, :CHH]
    thi_ref[...] = e[:, CHH:] * w_row[:, CHH:]

    @pl.when(r == 0)
    def _():
        elo_ref[...] = e[:, :CHH]
        ehi_ref[...] = e[:, CHH:]


_prep = pl.pallas_call(
    _prep_body,
    grid=(N_RELATIONS - 1,),
    in_specs=[
        pl.BlockSpec((N_ENTITIES, CHANNEL), lambda r: (0, 0)),
        pl.BlockSpec((N_RELATIONS - 1, CHANNEL), lambda r: (0, 0)),
    ],
    out_specs=[
        pl.BlockSpec((N_ENTITIES, CHH), lambda r: (r, 0)),
        pl.BlockSpec((N_ENTITIES, CHH), lambda r: (r, 0)),
        pl.BlockSpec((N_ENTITIES, CHH), lambda r: (0, 0)),
        pl.BlockSpec((N_ENTITIES, CHH), lambda r: (0, 0)),
    ],
    out_shape=[
        jax.ShapeDtypeStruct(((N_RELATIONS - 1) * N_ENTITIES, CHH),
                             jnp.float32),
        jax.ShapeDtypeStruct(((N_RELATIONS - 1) * N_ENTITIES, CHH),
                             jnp.float32),
        jax.ShapeDtypeStruct((N_ENTITIES, CHH), jnp.float32),
        jax.ShapeDtypeStruct((N_ENTITIES, CHH), jnp.float32),
    ],
)


_RB = 1000


def _combine_body(kglo_ref, kghi_ref, cntt_ref, uslo_ref, ushi_ref, usr_ref,
                  lat_ref, att_ref, w_ref, ent_o, usr_o, cor_o):
    sums = jnp.concatenate([kglo_ref[0] + kglo_ref[1],
                            kghi_ref[0] + kghi_ref[1]], axis=1)
    cnt = jnp.sum(cntt_ref[...], axis=1)
    ent_agg = sums / jnp.maximum(cnt, 1.0)[:, None]
    en = jnp.sqrt(jnp.sum(ent_agg * ent_agg, axis=1, keepdims=True))
    ent_o[...] = ent_agg / jnp.maximum(en, 1e-12)

    usr = usr_ref[...]
    logits = lax.dot_general(usr, lat_ref[...], (((1,), (1,)), ((), ())),
                             preferred_element_type=jnp.float32)
    score = jax.nn.softmax(logits, axis=1)
    att = att_ref[...]
    dw = lax.dot_general(jax.nn.softmax(att, axis=-1), w_ref[...],
                         (((1,), (0,)), ((), ())),
                         preferred_element_type=jnp.float32)
    factor = 1.0 + lax.dot_general(score, dw, (((1,), (0,)), ((), ())),
                                   preferred_element_type=jnp.float32)
    us = jnp.concatenate([uslo_ref[0] + uslo_ref[1],
                          ushi_ref[0] + ushi_ref[1]], axis=1)
    usr_agg = us * factor
    un = jnp.sqrt(jnp.sum(usr_agg * usr_agg, axis=1, keepdims=True))
    usr_o[...] = usr_agg / jnp.maximum(un, 1e-12)

    @pl.when(pl.program_id(0) == 0)
    def _():
        gram = lax.dot_general(att, att, (((0,), (0,)), ((), ())),
                               preferred_element_type=jnp.float32)
        ttl = jnp.exp(jnp.sum(gram, axis=1) / TEMP)
        colsq = jnp.sum(att * att, axis=0)
        pos = jnp.exp((colsq / colsq) / TEMP)
        cor_o[...] = (-jnp.sum(jnp.log(pos / ttl))).reshape(1, 1)


_combine = pl.pallas_call(
    _combine_body,
    grid=(N_ENTITIES // _RB,),
    in_specs=[
        pl.BlockSpec((NC, _RB, CHH), lambda i: (0, i, 0)),
        pl.BlockSpec((NC, _RB, CHH), lambda i: (0, i, 0)),
        pl.BlockSpec((_RB, NW), lambda i: (i, 0)),
        pl.BlockSpec((NC, _RB, CHH), lambda i: (0, i, 0)),
        pl.BlockSpec((NC, _RB, CHH), lambda i: (0, i, 0)),
        pl.BlockSpec((_RB, CHANNEL), lambda i: (i, 0)),
        pl.BlockSpec((N_FACTORS, CHANNEL), lambda i: (0, 0)),
        pl.BlockSpec((N_FACTORS, N_RELATIONS - 1), lambda i: (0, 0)),
        pl.BlockSpec((N_RELATIONS - 1, CHANNEL), lambda i: (0, 0)),
    ],
    out_specs=[
        pl.BlockSpec((_RB, CHANNEL), lambda i: (i, 0)),
        pl.BlockSpec((_RB, CHANNEL), lambda i: (i, 0)),
        pl.BlockSpec((1, 1), lambda i: (0, 0)),
    ],
    out_shape=[
        jax.ShapeDtypeStruct((N_ENTITIES, CHANNEL), jnp.float32),
        jax.ShapeDtypeStruct((N_USERS, CHANNEL), jnp.float32),
        jax.ShapeDtypeStruct((1, 1), jnp.float32),
    ],
)


def _pad_idx(x, fill=None):
    if fill is None:
        pad = N_ENTITIES + (jnp.arange(E_PAD - E_TOTAL, dtype=x.dtype)
                            % TRASH_N)
    else:
        pad = jnp.full((E_PAD - E_TOTAL,), fill, x.dtype)
    return jnp.concatenate([x, pad]).reshape(E_PAD // CH, CH)


def kernel(user_emb, entity_emb, latent_emb, edge_index, edge_type,
           interact_indices, interact_values, weight, disen_weight_att):
    head = _pad_idx(edge_index[0])
    kg_idx = _pad_idx((edge_type - 1) * N_ENTITIES + edge_index[1], 0)
    row = _pad_idx(interact_indices[0])
    col = _pad_idx(interact_indices[1], 0)
    val = _pad_idx(interact_values, 0.0)

    ent_cur, usr_cur = entity_emb, user_emb
    ent_res, usr_res = entity_emb, user_emb
    cor = None
    cnt_t = None
    for hop in range(2):
        t_lo, t_hi, e_lo, e_hi = _prep(ent_cur, weight)
        kg_lo, kg_hi, cnt_p, us_lo, us_hi = _sc_hop_build(hop == 0)(
            t_lo, t_hi, e_lo, e_hi, kg_idx, head, col, row, val)
        kg_lo = kg_lo.reshape(NC, N_ENTITIES, CHH)
        kg_hi = kg_hi.reshape(NC, N_ENTITIES, CHH)
        us_lo = us_lo.reshape(NC, N_USERS, CHH)
        us_hi = us_hi.reshape(NC, N_USERS, CHH)
        if hop == 0:
            cnt_t = cnt_p.reshape(NW, N_ENTITIES).T
        ent_cur, usr_cur, cor11 = _combine(kg_lo, kg_hi, cnt_t, us_lo, us_hi,
                                           usr_cur, latent_emb,
                                           disen_weight_att, weight)
        if cor is None:
            cor = cor11[0, 0]
        ent_res = ent_res + ent_cur
        usr_res = usr_res + usr_cur
    return (ent_res, usr_res, cor)
